# baseline (speedup 1.0000x reference)
import functools

import numpy as np

import jax
import jax.numpy as jnp
from jax import lax
from jax.experimental import pallas as pl
from jax.experimental.pallas import tpu as pltpu
from jax.experimental.pallas import tpu_sc as plsc

_NW = 32

_CM_BITS = (
    "0111010101010110111111011110010101100111000001001011111011010100"
    "00111000100100100001110111011011"
)
_POS_COLS = (0, 1)
_NEG_COLS = (222, 223)
_FIXED_CHW = (96, 224, 224)


def _runs(cols):
    cols = sorted(cols)
    out = []
    for c in cols:
        if out and c == out[-1][0] + out[-1][1]:
            out[-1] = (out[-1][0], out[-1][1] + 1)
        else:
            out.append((c, 1))
    return out


def _pack_bits(bits):
    words = []
    for k in range(0, len(bits), 32):
        v = 0
        for j, b in enumerate(bits[k:k + 32]):
            v |= int(bool(b)) << j
        if v >= 1 << 31:
            v -= 1 << 32
        words.append(v)
    return words


def _cmval(cm_words, c):
    acc = jnp.int32(cm_words[0])
    for k in range(1, len(cm_words)):
        acc = jnp.where(c >= 32 * k, jnp.int32(cm_words[k]), acc)
    return (acc >> (c % 32)) & 1


def _make_tc_out(N, C, H, W, cm_words, maj1, edge_runs):
    rows = N * C * H
    G = N * C
    ew = sum(wd for (_, wd, _) in edge_runs)

    def body(lst_ref, gui_ref, slab_ref, out_ref, buf, sem):
        g = pl.program_id(0)
        srcs = (lst_ref, gui_ref)

        def start(i, s):
            v = _cmval(cm_words, i % C)
            for val in (1, 0):
                @pl.when(v == val)
                def _():
                    src = srcs[maj1 if val else 1 - maj1]
                    pltpu.make_async_copy(
                        src.at[pl.ds(i * H, H), :], buf.at[s], sem.at[s]
                    ).start()

        s = g % 2

        @pl.when(g == 0)
        def _():
            start(g, s)

        @pl.when(g + 1 < G)
        def _():
            start(g + 1, 1 - s)

        pltpu.make_async_copy(
            lst_ref.at[pl.ds(g * H, H), :], buf.at[s], sem.at[s]).wait()
        out_ref[...] = buf[s]
        off = 0
        for (a, wd, _) in edge_runs:
            out_ref[:, pl.ds(a, wd)] = slab_ref[0, :, pl.ds(off, wd)]
            off += wd

    return pl.pallas_call(
        body,
        grid=(G,),
        in_specs=[
            pl.BlockSpec(memory_space=pl.ANY),
            pl.BlockSpec(memory_space=pl.ANY),
            pl.BlockSpec((1, H, ew), lambda g: (g, 0, 0)),
        ],
        out_specs=pl.BlockSpec((H, W), lambda g: (g, 0)),
        out_shape=jax.ShapeDtypeStruct((rows, W), jnp.float32),
        scratch_shapes=[
            pltpu.VMEM((2, H, W), jnp.float32),
            pltpu.SemaphoreType.DMA((2,)),
        ],
    )


def _make_sc_out(N, C, H, W, cm_words, maj1):
    rows = N * C * H
    G = N * C
    per_w = G // _NW
    mesh = plsc.VectorSubcoreMesh(
        core_axis_name="c", subcore_axis_name="s", num_cores=2, num_subcores=16)

    @functools.partial(
        pl.kernel, mesh=mesh,
        out_type=jax.ShapeDtypeStruct((rows, W), jnp.float32),
        scratch_types=[
            pltpu.VMEM((H, W), jnp.float32),
            pltpu.VMEM((H, W), jnp.float32),
            pltpu.SemaphoreType.DMA, pltpu.SemaphoreType.DMA,
            pltpu.SemaphoreType.DMA, pltpu.SemaphoreType.DMA,
        ],
    )
    def sc_out(lst_hbm, gui_hbm, out_hbm, buf0, buf1, si0, si1, so0, so1):
        bufs = (buf0, buf1)
        sin = (si0, si1)
        sout = (so0, so1)
        srcs = (lst_hbm, gui_hbm)
        wid = lax.axis_index("s") * 2 + lax.axis_index("c")
        img0 = wid * per_w

        def gather_start(i, s):
            img = img0 + i
            v = _cmval(cm_words, img % C)
            for val in (1, 0):
                @pl.when(v == val)
                def _():
                    src = srcs[maj1 if val else 1 - maj1]
                    pltpu.make_async_copy(
                        src.at[pl.ds(img * H, H), :], bufs[s], sin[s]).start()

        def gather_wait(i, s):
            pltpu.make_async_copy(
                lst_hbm.at[pl.ds((img0 + i) * H, H), :], bufs[s], sin[s]).wait()

        def scatter(i, s):
            return pltpu.make_async_copy(
                bufs[s], out_hbm.at[pl.ds((img0 + i) * H, H), :], sout[s])

        gather_start(0, 0)
        for i in range(per_w):
            s = i % 2
            if i + 1 < per_w:
                if i >= 1:
                    scatter(i - 1, 1 - s).wait()
                gather_start(i + 1, 1 - s)
            gather_wait(i, s)
            scatter(i, s).start()
        if per_w >= 2:
            scatter(per_w - 2, per_w % 2).wait()
        scatter(per_w - 1, (per_w - 1) % 2).wait()

    return sc_out


def _select_fallback(lst, gui):
    N, C, H, W = lst.shape
    mk = jax.random.key(42)
    kc, ks = jax.random.split(mk)
    cm = jax.random.randint(kc, (C,), 0, 2).astype(jnp.uint8).astype(bool)
    spatial = jax.random.randint(ks, (H,), 0, 2)
    neg_hit = jnp.zeros((W,), bool).at[jnp.bitwise_not(spatial) % W].set(True)
    pos_hit = jnp.zeros((W,), bool).at[spatial % W].set(True)
    take = pos_hit[None, :] | (~neg_hit[None, :] & cm[:, None])
    mask = take.astype(jnp.float32).reshape(C, 1, W)

    def body(m_ref, a_ref, b_ref, o1_ref, o2_ref):
        m = (m_ref[...] != 0.0)[None]
        a = a_ref[...]
        b = b_ref[...]
        o1_ref[...] = jnp.where(m, b, a)
        o2_ref[...] = jnp.where(m, a, b)

    CB = 8
    while C % CB:
        CB //= 2
    grid = (N, C // CB)
    data_spec = pl.BlockSpec((1, CB, H, W), lambda n, c: (n, c, 0, 0))
    mask_spec = pl.BlockSpec((CB, 1, W), lambda n, c: (c, 0, 0))
    return tuple(pl.pallas_call(
        body,
        grid=grid,
        in_specs=[mask_spec, data_spec, data_spec],
        out_specs=[data_spec, data_spec],
        out_shape=[
            jax.ShapeDtypeStruct(lst.shape, lst.dtype),
            jax.ShapeDtypeStruct(gui.shape, gui.dtype),
        ],
    )(mask, lst, gui))


def kernel(lst, gui):
    N, C, H, W = lst.shape
    if (C, H, W) != _FIXED_CHW or (N * C) % _NW or lst.dtype != jnp.float32:
        return _select_fallback(lst, gui)

    cm_words = _pack_bits([b == "1" for b in _CM_BITS])
    pos_runs = _runs(_POS_COLS)
    neg_runs = _runs(_NEG_COLS)
    lst_edges = [(a, w, 1) for (a, w) in pos_runs] + [(a, w, 0) for (a, w) in neg_runs]
    gui_edges = [(a, w, 0) for (a, w) in pos_runs] + [(a, w, 1) for (a, w) in neg_runs]

    rows = N * C * H
    G = N * C
    lst2 = lst.reshape(rows, W)
    gui2 = gui.reshape(rows, W)
    srcs2 = (lst2, gui2)

    slab = jnp.concatenate(
        [srcs2[src][:, a:a + wd] for (a, wd, src) in lst_edges], axis=1
    ).reshape(G, H, sum(wd for (_, wd, _) in lst_edges))

    sc_gui = _make_sc_out(N, C, H, W, cm_words, 0)(lst2, gui2)
    out_lst = _make_tc_out(N, C, H, W, cm_words, 1, lst_edges)(lst2, gui2, slab)
    out_gui = sc_gui
    for (a, wd, src) in gui_edges:
        out_gui = out_gui.at[:, a:a + wd].set(srcs2[src][:, a:a + wd])
    return (out_lst.reshape(N, C, H, W), out_gui.reshape(N, C, H, W))


# speedup vs baseline: 8.1043x; 8.1043x over previous
import jax
import jax.numpy as jnp
from jax.experimental import pallas as pl


def _select_body(m_ref, a_ref, b_ref, o1_ref, o2_ref):
    m = (m_ref[...] != 0.0)[None]
    a = a_ref[...]
    b = b_ref[...]
    o1_ref[...] = jnp.where(m, b, a)
    o2_ref[...] = jnp.where(m, a, b)


def _masks(C, H, W):
    mk = jax.random.key(42)
    kc, ks = jax.random.split(mk)
    cm = jax.random.randint(kc, (C,), 0, 2).astype(jnp.uint8).astype(bool)
    spatial = jax.random.randint(ks, (H,), 0, 2)
    neg_idx = jnp.bitwise_not(spatial) % W
    pos_idx = spatial % W
    neg_hit = jnp.zeros((W,), dtype=bool).at[neg_idx].set(True)
    pos_hit = jnp.zeros((W,), dtype=bool).at[pos_idx].set(True)
    take_gui = pos_hit[None, :] | (~neg_hit[None, :] & cm[:, None])
    return take_gui


def kernel(lst, gui):
    N, C, H, W = lst.shape
    mask = _masks(C, H, W).astype(jnp.float32).reshape(C, 1, W)

    CB = 16
    while C % CB:
        CB //= 2
    grid = (N, C // CB)

    data_spec = pl.BlockSpec((1, CB, H, W), lambda n, c: (n, c, 0, 0))
    mask_spec = pl.BlockSpec((CB, 1, W), lambda n, c: (c, 0, 0))

    out_lst, out_gui = pl.pallas_call(
        _select_body,
        grid=grid,
        in_specs=[mask_spec, data_spec, data_spec],
        out_specs=[data_spec, data_spec],
        out_shape=[
            jax.ShapeDtypeStruct(lst.shape, lst.dtype),
            jax.ShapeDtypeStruct(gui.shape, gui.dtype),
        ],
    )(mask, lst, gui)
    return (out_lst, out_gui)


# speedup vs baseline: 8.1574x; 1.0065x over previous
import jax
import jax.numpy as jnp
from jax.experimental import pallas as pl


def _select_body(m_ref, a_ref, b_ref, o1_ref, o2_ref):
    m = (m_ref[...] != 0.0)[None]
    a = a_ref[...]
    b = b_ref[...]
    o1_ref[...] = jnp.where(m, b, a)
    o2_ref[...] = jnp.where(m, a, b)


def _masks(C, H, W):
    mk = jax.random.key(42)
    kc, ks = jax.random.split(mk)
    cm = jax.random.randint(kc, (C,), 0, 2).astype(jnp.uint8).astype(bool)
    spatial = jax.random.randint(ks, (H,), 0, 2)
    neg_idx = jnp.bitwise_not(spatial) % W
    pos_idx = spatial % W
    neg_hit = jnp.zeros((W,), dtype=bool).at[neg_idx].set(True)
    pos_hit = jnp.zeros((W,), dtype=bool).at[pos_idx].set(True)
    take_gui = pos_hit[None, :] | (~neg_hit[None, :] & cm[:, None])
    return take_gui


def kernel(lst, gui):
    N, C, H, W = lst.shape
    mask = _masks(C, H, W).astype(jnp.float32).reshape(C, 1, W)

    CB = 32
    while C % CB:
        CB //= 2
    grid = (N, C // CB)

    data_spec = pl.BlockSpec((1, CB, H, W), lambda n, c: (n, c, 0, 0))
    mask_spec = pl.BlockSpec((CB, 1, W), lambda n, c: (c, 0, 0))

    out_lst, out_gui = pl.pallas_call(
        _select_body,
        grid=grid,
        in_specs=[mask_spec, data_spec, data_spec],
        out_specs=[data_spec, data_spec],
        out_shape=[
            jax.ShapeDtypeStruct(lst.shape, lst.dtype),
            jax.ShapeDtypeStruct(gui.shape, gui.dtype),
        ],
    )(mask, lst, gui)
    return (out_lst, out_gui)
